# Initial kernel scaffold; baseline (speedup 1.0000x reference)
import functools

import jax
import jax.numpy as jnp
from jax import lax
from jax.experimental import pallas as pl
from jax.experimental.pallas import tpu as pltpu
from jax.experimental.pallas import tpu_sc as plsc

NUM_FIELDS = 26
VOCAB = 100000
EMB = 16

_NC = 2
_NS = 16
_NW = _NC * _NS

_LANES = 16
_IDXW = 128
_JROWS = 13
_CHUNK = _JROWS * _IDXW


@functools.partial(jax.jit, static_argnums=(2, 3))
def _sc_gather(tab, idx2d, n_rows, chunks_per_w):
    mesh = plsc.VectorSubcoreMesh(core_axis_name="c", subcore_axis_name="s")
    rows_per_w = n_rows // _NW
    irows_per_w = rows_per_w // _IDXW

    @functools.partial(
        pl.kernel,
        out_type=jax.ShapeDtypeStruct((n_rows, EMB), jnp.float32),
        mesh=mesh,
        scratch_types=[
            pltpu.VMEM((_JROWS, _IDXW), jnp.int32),
            pltpu.VMEM((_CHUNK, EMB), jnp.float32),
            pltpu.SemaphoreType.DMA,
        ],
    )
    def k(tab_hbm, idx_hbm, out_hbm, idx_v, rows_v, sem):
        wid = lax.axis_index("c") * _NS + lax.axis_index("s")
        irow_base = wid * irows_per_w
        row_base = wid * rows_per_w

        lane = lax.iota(jnp.int32, _LANES)
        offs = [((lane + (16 * m) % 26) % 26) * VOCAB for m in range(_JROWS)]

        def body(g, carry):
            pltpu.sync_copy(idx_hbm.at[pl.ds(irow_base + g * _JROWS, _JROWS)],
                            idx_v)
            for j in range(_JROWS):
                for kk in range(_IDXW // _LANES):
                    v = (8 * j + kk) % _JROWS
                    sl = pl.ds(kk * _LANES, _LANES)
                    idx_v[j, sl] = idx_v[j, sl] + offs[v]
            cps = [
                pltpu.async_copy(
                    tab_hbm.at[idx_v.at[j]],
                    rows_v.at[pl.ds(j * _IDXW, _IDXW)],
                    sem,
                )
                for j in range(_JROWS)
            ]
            for c in cps:
                c.wait()
            pltpu.sync_copy(
                rows_v,
                out_hbm.at[pl.ds(row_base + g * _CHUNK, _CHUNK)])
            return carry

        lax.fori_loop(0, chunks_per_w, body, None)

    return k(tab, idx2d)


def kernel(x, tables):
    B, S, F = x.shape
    Fv, V, E = tables.shape
    n_rows = B * S * F
    chunks_per_w = n_rows // (_NW * _CHUNK)
    assert chunks_per_w * _NW * _CHUNK == n_rows
    idx2d = x.reshape(n_rows // _IDXW, _IDXW).astype(jnp.int32)
    tab = tables.reshape(Fv * V, E)
    out = _sc_gather(tab, idx2d, n_rows, chunks_per_w)
    return out.reshape(B, S, F * E)


# speedup vs baseline: 5.0855x; 5.0855x over previous
import functools

import jax
import jax.numpy as jnp
from jax import lax
from jax.experimental import pallas as pl
from jax.experimental.pallas import tpu as pltpu
from jax.experimental.pallas import tpu_sc as plsc

NUM_FIELDS = 26
VOCAB = 100000
EMB = 16

_NC = 2
_NS = 16
_NW = _NC * _NS

_LANES = 16
_IDXW = 128
_JROWS = 13
_CHUNK = _JROWS * _IDXW


@functools.partial(jax.jit, static_argnums=(2, 3))
def _sc_gather(tab, idx1d, n_rows, chunks_per_w):
    mesh = plsc.VectorSubcoreMesh(core_axis_name="c", subcore_axis_name="s")
    rows_per_w = n_rows // _NW
    nvec = _CHUNK // _LANES

    @functools.partial(
        pl.kernel,
        out_type=jax.ShapeDtypeStruct((n_rows, EMB), jnp.float32),
        mesh=mesh,
        scratch_types=[
            pltpu.VMEM((_CHUNK,), jnp.int32),
            pltpu.VMEM((_CHUNK, EMB), jnp.float32),
            pltpu.SemaphoreType.DMA,
        ],
        compiler_params=pltpu.CompilerParams(use_tc_tiling_on_sc=False),
    )
    def k(tab_hbm, idx_hbm, out_hbm, idx_v, rows_v, sem):
        wid = lax.axis_index("c") * _NS + lax.axis_index("s")
        row_base = wid * rows_per_w

        lane = lax.iota(jnp.int32, _LANES)
        offs = [((lane + (16 * m) % 26) % 26) * VOCAB for m in range(_JROWS)]

        def body(g, carry):
            start = row_base + g * _CHUNK
            pltpu.sync_copy(idx_hbm.at[pl.ds(start, _CHUNK)], idx_v)
            for v in range(nvec):
                sl = pl.ds(v * _LANES, _LANES)
                idx_v[sl] = idx_v[sl] + offs[v % _JROWS]
            cps = [
                pltpu.async_copy(
                    tab_hbm.at[idx_v.at[pl.ds(j * _IDXW, _IDXW)]],
                    rows_v.at[pl.ds(j * _IDXW, _IDXW)],
                    sem,
                )
                for j in range(_JROWS)
            ]
            for c in cps:
                c.wait()
            pltpu.sync_copy(rows_v, out_hbm.at[pl.ds(start, _CHUNK)])
            return carry

        lax.fori_loop(0, chunks_per_w, body, None)

    return k(tab, idx1d)


def kernel(x, tables):
    B, S, F = x.shape
    Fv, V, E = tables.shape
    n_rows = B * S * F
    chunks_per_w = n_rows // (_NW * _CHUNK)
    assert chunks_per_w * _NW * _CHUNK == n_rows
    idx1d = x.reshape(n_rows).astype(jnp.int32)
    tab = tables.reshape(Fv * V, E)
    out = _sc_gather(tab, idx1d, n_rows, chunks_per_w)
    return out.reshape(B, S, F * E)


# speedup vs baseline: 5.2941x; 1.0410x over previous
import functools

import jax
import jax.numpy as jnp
from jax import lax
from jax.experimental import pallas as pl
from jax.experimental.pallas import tpu as pltpu
from jax.experimental.pallas import tpu_sc as plsc

NUM_FIELDS = 26
VOCAB = 100000
EMB = 16

_NC = 2
_NS = 16
_NW = _NC * _NS

_LANES = 16
_IDXW = 128
_JROWS = 13
_CHUNK = _JROWS * _IDXW


@functools.partial(jax.jit, static_argnums=(2, 3))
def _sc_gather(tab, idx1d, n_rows, chunks_per_w):
    mesh = plsc.VectorSubcoreMesh(core_axis_name="c", subcore_axis_name="s")
    rows_per_w = n_rows // _NW
    nvec = _CHUNK // _LANES

    @functools.partial(
        pl.kernel,
        out_type=jax.ShapeDtypeStruct((n_rows, EMB), jnp.float32),
        mesh=mesh,
        scratch_types=[
            pltpu.VMEM((_CHUNK,), jnp.int32),
            pltpu.VMEM((_CHUNK,), jnp.int32),
            pltpu.VMEM((_CHUNK, EMB), jnp.float32),
            pltpu.VMEM((_CHUNK, EMB), jnp.float32),
            pltpu.SemaphoreType.DMA,
            pltpu.SemaphoreType.DMA,
            pltpu.SemaphoreType.DMA,
            pltpu.SemaphoreType.DMA,
            pltpu.SemaphoreType.DMA,
        ],
        compiler_params=pltpu.CompilerParams(use_tc_tiling_on_sc=False),
    )
    def k(tab_hbm, idx_hbm, out_hbm, idx0, idx1, rows0, rows1,
          si0, si1, sg, so0, so1):
        wid = lax.axis_index("c") * _NS + lax.axis_index("s")
        row_base = wid * rows_per_w
        G = chunks_per_w
        idx_v = [idx0, idx1]
        rows_v = [rows0, rows1]
        sem_i = [si0, si1]
        sem_o = [so0, so1]

        lane = lax.iota(jnp.int32, _LANES)
        offs = [((lane + (16 * m) % 26) % 26) * VOCAB for m in range(_JROWS)]

        def idx_cp(c, b):
            return pltpu.make_async_copy(
                idx_hbm.at[pl.ds(row_base + c * _CHUNK, _CHUNK)],
                idx_v[b], sem_i[b])

        def out_cp(c, b):
            return pltpu.make_async_copy(
                rows_v[b], out_hbm.at[pl.ds(row_base + c * _CHUNK, _CHUNK)],
                sem_o[b])

        def gather_cps(b):
            return [pltpu.make_async_copy(
                        tab_hbm.at[idx_v[b].at[pl.ds(j * _IDXW, _IDXW)]],
                        rows_v[b].at[pl.ds(j * _IDXW, _IDXW)], sg)
                    for j in range(_JROWS)]

        def add_offsets(b):
            for v in range(nvec):
                sl = pl.ds(v * _LANES, _LANES)
                idx_v[b][sl] = idx_v[b][sl] + offs[v % _JROWS]

        idx_cp(0, 0).start()
        idx_cp(0, 0).wait()
        add_offsets(0)
        for cp in gather_cps(0):
            cp.start()
        idx_cp(1, 1).start()

        def step(c, b):
            @pl.when(c + 1 < G)
            def _():
                idx_cp(c + 1, 1 - b).wait()
                add_offsets(1 - b)
            for cp in gather_cps(b):
                cp.wait()
            out_cp(c, b).start()
            @pl.when(c > 0)
            def _():
                out_cp(c - 1, 1 - b).wait()
            @pl.when(c + 1 < G)
            def _():
                for cp in gather_cps(1 - b):
                    cp.start()
            @pl.when(c + 2 < G)
            def _():
                idx_cp(c + 2, b).start()

        def body(h, carry):
            step(2 * h, 0)
            step(2 * h + 1, 1)
            return carry

        lax.fori_loop(0, G // 2, body, None)
        out_cp(G - 1, (G - 1) % 2).wait()

    return k(tab, idx1d)


def kernel(x, tables):
    B, S, F = x.shape
    Fv, V, E = tables.shape
    n_rows = B * S * F
    chunks_per_w = n_rows // (_NW * _CHUNK)
    assert chunks_per_w * _NW * _CHUNK == n_rows
    idx1d = x.reshape(n_rows).astype(jnp.int32)
    tab = tables.reshape(Fv * V, E)
    out = _sc_gather(tab, idx1d, n_rows, chunks_per_w)
    return out.reshape(B, S, F * E)


# speedup vs baseline: 5.2976x; 1.0007x over previous
import functools

import jax
import jax.numpy as jnp
from jax import lax
from jax.experimental import pallas as pl
from jax.experimental.pallas import tpu as pltpu
from jax.experimental.pallas import tpu_sc as plsc

NUM_FIELDS = 26
VOCAB = 100000
EMB = 16

_NC = 2
_NS = 16
_NW = _NC * _NS

_LANES = 16
_IDXW = 128
_JROWS = 13
_CHUNK = _JROWS * _IDXW


@functools.partial(jax.jit, static_argnums=(2, 3))
def _sc_gather(tab, idx1d, n_rows, chunks_per_w):
    mesh = plsc.VectorSubcoreMesh(core_axis_name="c", subcore_axis_name="s")
    rows_per_w = n_rows // _NW
    nvec = _CHUNK // _LANES

    @functools.partial(
        pl.kernel,
        out_type=jax.ShapeDtypeStruct((n_rows, EMB), jnp.float32),
        mesh=mesh,
        scratch_types=[
            pltpu.VMEM((_CHUNK,), jnp.int32),
            pltpu.VMEM((_CHUNK,), jnp.int32),
            pltpu.VMEM((_CHUNK, EMB), jnp.float32),
            pltpu.VMEM((_CHUNK, EMB), jnp.float32),
            pltpu.SemaphoreType.DMA,
            pltpu.SemaphoreType.DMA,
            pltpu.SemaphoreType.DMA,
            pltpu.SemaphoreType.DMA,
            pltpu.SemaphoreType.DMA,
        ],
        compiler_params=pltpu.CompilerParams(use_tc_tiling_on_sc=False),
    )
    def k(tab_hbm, idx_hbm, out_hbm, idx0, idx1, rows0, rows1,
          si0, si1, sg, so0, so1):
        wid = lax.axis_index("c") * _NS + lax.axis_index("s")
        row_base = wid * rows_per_w
        G = chunks_per_w
        idx_v = [idx0, idx1]
        rows_v = [rows0, rows1]
        sem_i = [si0, si1]
        sem_o = [so0, so1]

        lane = lax.iota(jnp.int32, _LANES)
        offs = [((lane + (16 * m) % 26) % 26) * VOCAB for m in range(_JROWS)]

        def idx_cp(c, b):
            return pltpu.make_async_copy(
                idx_hbm.at[pl.ds(row_base + c * _CHUNK, _CHUNK)],
                idx_v[b], sem_i[b])

        def out_cp(c, b):
            return pltpu.make_async_copy(
                rows_v[b], out_hbm.at[pl.ds(row_base + c * _CHUNK, _CHUNK)],
                sem_o[b])

        def gather_cps(b):
            return [pltpu.make_async_copy(tab_hbm.at[idx_v[b]], rows_v[b], sg)]

        def add_offsets(b):
            for v in range(nvec):
                sl = pl.ds(v * _LANES, _LANES)
                idx_v[b][sl] = idx_v[b][sl] + offs[v % _JROWS]

        idx_cp(0, 0).start()
        idx_cp(0, 0).wait()
        add_offsets(0)
        for cp in gather_cps(0):
            cp.start()
        idx_cp(1, 1).start()

        def step(c, b):
            @pl.when(c + 1 < G)
            def _():
                idx_cp(c + 1, 1 - b).wait()
                add_offsets(1 - b)
            for cp in gather_cps(b):
                cp.wait()
            out_cp(c, b).start()
            @pl.when(c > 0)
            def _():
                out_cp(c - 1, 1 - b).wait()
            @pl.when(c + 1 < G)
            def _():
                for cp in gather_cps(1 - b):
                    cp.start()
            @pl.when(c + 2 < G)
            def _():
                idx_cp(c + 2, b).start()

        def body(h, carry):
            step(2 * h, 0)
            step(2 * h + 1, 1)
            return carry

        lax.fori_loop(0, G // 2, body, None)
        out_cp(G - 1, (G - 1) % 2).wait()

    return k(tab, idx1d)


def kernel(x, tables):
    B, S, F = x.shape
    Fv, V, E = tables.shape
    n_rows = B * S * F
    chunks_per_w = n_rows // (_NW * _CHUNK)
    assert chunks_per_w * _NW * _CHUNK == n_rows
    idx1d = x.reshape(n_rows).astype(jnp.int32)
    tab = tables.reshape(Fv * V, E)
    out = _sc_gather(tab, idx1d, n_rows, chunks_per_w)
    return out.reshape(B, S, F * E)


# speedup vs baseline: 5.3010x; 1.0006x over previous
import functools

import jax
import jax.numpy as jnp
from jax import lax
from jax.experimental import pallas as pl
from jax.experimental.pallas import tpu as pltpu
from jax.experimental.pallas import tpu_sc as plsc

NUM_FIELDS = 26
VOCAB = 100000
EMB = 16

_NC = 2
_NS = 16
_NW = _NC * _NS

_LANES = 16
_IDXW = 128
_JROWS = 13
_CHUNK = _JROWS * _IDXW


@functools.partial(jax.jit, static_argnums=(2, 3, 4))
def _sc_gather(tab3d, idx1d, n_rows, chunks_per_w, out_shape):
    mesh = plsc.VectorSubcoreMesh(core_axis_name="c", subcore_axis_name="s")
    rows_per_w = n_rows // _NW
    nvec = _CHUNK // _LANES

    @functools.partial(
        pl.kernel,
        out_type=jax.ShapeDtypeStruct((n_rows, EMB), jnp.float32),
        mesh=mesh,
        scratch_types=[
            pltpu.VMEM((_CHUNK,), jnp.int32),
            pltpu.VMEM((_CHUNK,), jnp.int32),
            pltpu.VMEM((_CHUNK, EMB), jnp.float32),
            pltpu.VMEM((_CHUNK, EMB), jnp.float32),
            pltpu.SemaphoreType.DMA,
            pltpu.SemaphoreType.DMA,
            pltpu.SemaphoreType.DMA,
            pltpu.SemaphoreType.DMA,
            pltpu.SemaphoreType.DMA,
        ],
        compiler_params=pltpu.CompilerParams(use_tc_tiling_on_sc=False),
    )
    def k(tab3d_hbm, idx_hbm, out_hbm, idx0, idx1, rows0, rows1,
          si0, si1, sg, so0, so1):
        tab_hbm = tab3d_hbm.at[0]
        wid = lax.axis_index("c") * _NS + lax.axis_index("s")
        row_base = wid * rows_per_w
        G = chunks_per_w
        idx_v = [idx0, idx1]
        rows_v = [rows0, rows1]
        sem_i = [si0, si1]
        sem_o = [so0, so1]

        lane = lax.iota(jnp.int32, _LANES)
        offs = [((lane + (16 * m) % 26) % 26) * VOCAB for m in range(_JROWS)]

        def idx_cp(c, b):
            return pltpu.make_async_copy(
                idx_hbm.at[pl.ds(row_base + c * _CHUNK, _CHUNK)],
                idx_v[b], sem_i[b])

        def out_cp(c, b):
            return pltpu.make_async_copy(
                rows_v[b], out_hbm.at[pl.ds(row_base + c * _CHUNK, _CHUNK)],
                sem_o[b])

        def gather_cps(b):
            return [pltpu.make_async_copy(tab_hbm.at[idx_v[b]], rows_v[b], sg)]

        def add_offsets(b):
            for v in range(nvec):
                sl = pl.ds(v * _LANES, _LANES)
                idx_v[b][sl] = idx_v[b][sl] + offs[v % _JROWS]

        idx_cp(0, 0).start()
        idx_cp(0, 0).wait()
        add_offsets(0)
        for cp in gather_cps(0):
            cp.start()
        idx_cp(1, 1).start()

        def step(c, b):
            @pl.when(c + 1 < G)
            def _():
                idx_cp(c + 1, 1 - b).wait()
                add_offsets(1 - b)
            for cp in gather_cps(b):
                cp.wait()
            out_cp(c, b).start()
            @pl.when(c > 0)
            def _():
                out_cp(c - 1, 1 - b).wait()
            @pl.when(c + 1 < G)
            def _():
                for cp in gather_cps(1 - b):
                    cp.start()
            @pl.when(c + 2 < G)
            def _():
                idx_cp(c + 2, b).start()

        def body(h, carry):
            step(2 * h, 0)
            step(2 * h + 1, 1)
            return carry

        lax.fori_loop(0, G // 2, body, None)
        out_cp(G - 1, (G - 1) % 2).wait()

    return k(tab3d, idx1d)


def kernel(x, tables):
    B, S, F = x.shape
    Fv, V, E = tables.shape
    n_rows = B * S * F
    chunks_per_w = n_rows // (_NW * _CHUNK)
    assert chunks_per_w * _NW * _CHUNK == n_rows
    idx1d = x.reshape(n_rows).astype(jnp.int32)
    out = _sc_gather(tables, idx1d, n_rows, chunks_per_w, (B, S, F * E))
    return out.reshape(B, S, F * E)


# speedup vs baseline: 5.3019x; 1.0002x over previous
import functools

import jax
import jax.numpy as jnp
from jax import lax
from jax.experimental import pallas as pl
from jax.experimental.pallas import tpu as pltpu
from jax.experimental.pallas import tpu_sc as plsc

NUM_FIELDS = 26
VOCAB = 100000
EMB = 16

_NC = 2
_NS = 16
_NW = _NC * _NS

_LANES = 16
_IDXW = 128
_JROWS = 13
_CHUNK = _JROWS * _IDXW


@functools.partial(jax.jit, static_argnums=(2, 3, 4))
def _sc_gather(tab3d, idx1d, n_rows, chunks_per_w, out_shape):
    mesh = plsc.VectorSubcoreMesh(core_axis_name="c", subcore_axis_name="s")
    rows_per_w = n_rows // _NW
    nvec = _CHUNK // _LANES

    @functools.partial(
        pl.kernel,
        out_type=jax.ShapeDtypeStruct((n_rows, EMB), jnp.float32),
        mesh=mesh,
        scratch_types=[
            pltpu.VMEM((_CHUNK,), jnp.int32),
            pltpu.VMEM((_CHUNK,), jnp.int32),
            pltpu.VMEM((_CHUNK, EMB), jnp.float32),
            pltpu.VMEM((_CHUNK, EMB), jnp.float32),
            pltpu.SemaphoreType.DMA,
            pltpu.SemaphoreType.DMA,
            pltpu.SemaphoreType.DMA,
            pltpu.SemaphoreType.DMA,
            pltpu.SemaphoreType.DMA,
        ],
        compiler_params=pltpu.CompilerParams(use_tc_tiling_on_sc=False),
    )
    def k(tab3d_hbm, idx_hbm, out_hbm, idx0, idx1, rows0, rows1,
          si0, si1, sg, so0, so1):
        tab_hbm = tab3d_hbm.at[0]
        wid = lax.axis_index("c") * _NS + lax.axis_index("s")
        row_base = wid * rows_per_w
        G = chunks_per_w
        idx_v = [idx0, idx1]
        rows_v = [rows0, rows1]
        sem_i = [si0, si1]
        sem_o = [so0, so1]

        lane = lax.iota(jnp.int32, _LANES)
        offs = [((lane + (16 * m) % 26) % 26) * VOCAB for m in range(_JROWS)]

        def idx_cp(c, b):
            return pltpu.make_async_copy(
                idx_hbm.at[pl.ds(row_base + c * _CHUNK, _CHUNK)],
                idx_v[b], sem_i[b])

        def out_cp(c, b):
            return pltpu.make_async_copy(
                rows_v[b], out_hbm.at[pl.ds(row_base + c * _CHUNK, _CHUNK)],
                sem_o[b])

        def gather_cp(b):
            return pltpu.make_async_copy(tab_hbm.at[idx_v[b]], rows_v[b], sg)

        def add_offsets(b):
            for v in range(nvec):
                sl = pl.ds(v * _LANES, _LANES)
                idx_v[b][sl] = idx_v[b][sl] + offs[v % _JROWS]

        idx_cp(0, 0).start()
        idx_cp(0, 0).wait()
        add_offsets(0)
        gather_cp(0).start()
        idx_cp(1, 1).start()

        def step(c, b):
            @pl.when(c + 1 < G)
            def _():
                idx_cp(c + 1, 1 - b).wait()
                add_offsets(1 - b)
            gather_cp(b).wait()
            out_cp(c, b).start()
            @pl.when(c > 0)
            def _():
                out_cp(c - 1, 1 - b).wait()
            @pl.when(c + 1 < G)
            def _():
                gather_cp(1 - b).start()
            @pl.when(c + 2 < G)
            def _():
                idx_cp(c + 2, b).start()

        def body(h, carry):
            step(2 * h, 0)
            step(2 * h + 1, 1)
            return carry

        lax.fori_loop(0, G // 2, body, None)
        out_cp(G - 1, (G - 1) % 2).wait()

    return k(tab3d, idx1d)


def kernel(x, tables):
    B, S, F = x.shape
    Fv, V, E = tables.shape
    n_rows = B * S * F
    chunks_per_w = n_rows // (_NW * _CHUNK)
    assert chunks_per_w * _NW * _CHUNK == n_rows
    idx1d = x.reshape(n_rows).astype(jnp.int32)
    out = _sc_gather(tables, idx1d, n_rows, chunks_per_w, (B, S, F * E))
    return out.reshape(B, S, F * E)
